# Initial kernel scaffold; baseline (speedup 1.0000x reference)
import jax
import jax.numpy as jnp
from jax.experimental import pallas as pl


def kernel(x, edge_index, edge_type, edge_attr, batch, W1, q1, k1, We1, e1, b1, W2, q2, k2, We2, e2, b2, W3, q3, k3, We3, e3, b3, lin_w, lin_b):
    raise NotImplementedError("write your pallas kernel here")


# speedup vs baseline: 11.3674x; 11.3674x over previous
import functools

import jax
import jax.numpy as jnp
from jax import lax
from jax.experimental import pallas as pl
from jax.experimental.pallas import tpu as pltpu
from jax.experimental.pallas import tpu_sc as plsc

N = 10000
E = 160000
D = 128
HEADS = 8
OUT = 128
R = 3
ACT = 16
G = 16
HID = HEADS * OUT

NC, NS, L = 2, 16, 16
NW = NC * NS
EBLK = 128
NBLK_TOTAL = E // EBLK
NPAD = 10240

_mesh = plsc.VectorSubcoreMesh(core_axis_name="c", subcore_axis_name="s",
                               num_cores=NC, num_subcores=NS)


def _mm_xrel(x, W):
    IN = x.shape[1]
    BN, BH = 1000, 512

    def body(x_ref, w_ref, o_ref):
        o_ref[...] = jnp.dot(x_ref[...], w_ref[0],
                             preferred_element_type=jnp.float32)[None]

    return pl.pallas_call(
        body,
        grid=(R, N // BN, HID // BH),
        in_specs=[
            pl.BlockSpec((BN, IN), lambda r, i, j: (i, 0)),
            pl.BlockSpec((1, IN, BH), lambda r, i, j: (r, 0, j)),
        ],
        out_specs=pl.BlockSpec((1, BN, BH), lambda r, i, j: (r, i, j)),
        out_shape=jax.ShapeDtypeStruct((R, N, HID), jnp.float32),
    )(x, W)


def _mm_wqk(W, qk):
    IN = W.shape[1]

    def body(w_ref, qk_ref, o_ref):
        o_ref[...] = jnp.dot(w_ref[0], qk_ref[...],
                             preferred_element_type=jnp.float32)[None]

    return pl.pallas_call(
        body,
        grid=(R,),
        in_specs=[
            pl.BlockSpec((1, IN, HID), lambda r: (r, 0, 0)),
            pl.BlockSpec((HID, 2 * HEADS), lambda r: (0, 0)),
        ],
        out_specs=pl.BlockSpec((1, IN, 2 * HEADS), lambda r: (r, 0, 0)),
        out_shape=jax.ShapeDtypeStruct((R, IN, 2 * HEADS), jnp.float32),
    )(W, qk)


def _mm_qktab(x, Wqk):
    IN = x.shape[1]
    BN = 1000

    def body(x_ref, w_ref, o_ref):
        v = jnp.dot(x_ref[...], w_ref[0],
                    preferred_element_type=jnp.float32)
        o_ref[...] = jnp.concatenate(
            [v, jnp.zeros((BN, 128 - 2 * HEADS), jnp.float32)], axis=1)[None]

    return pl.pallas_call(
        body,
        grid=(R, N // BN),
        in_specs=[
            pl.BlockSpec((BN, IN), lambda r, i: (i, 0)),
            pl.BlockSpec((1, IN, 2 * HEADS), lambda r, i: (r, 0, 0)),
        ],
        out_specs=pl.BlockSpec((1, BN, 128), lambda r, i: (r, i, 0)),
        out_shape=jax.ShapeDtypeStruct((R, N, 128), jnp.float32),
    )(x, Wqk)


def _cc_vec(We, e):
    def body(we_ref, e_ref, o_ref):
        c = jnp.dot(we_ref[...], e_ref[...],
                    preferred_element_type=jnp.float32)
        o_ref[:, 0:HEADS] = c
        o_ref[:, HEADS:2 * HEADS] = c

    return pl.pallas_call(
        body,
        out_shape=jax.ShapeDtypeStruct((1, 2 * HEADS), jnp.float32),
    )(We, e)


def _normalize(num8, den8, b):
    BN = 1024

    def body(n_ref, d_ref, b_ref, o_ref):
        i = pl.program_id(0)
        for h in range(HEADS):
            den = d_ref[h, pl.ds(i * BN, BN)][:, None] + 1e-16
            v = n_ref[h] / den + b_ref[h][None, :]
            o_ref[:, h * OUT:(h + 1) * OUT] = jnp.maximum(v, 0.0)

    return pl.pallas_call(
        body,
        grid=(NPAD // BN,),
        in_specs=[
            pl.BlockSpec((HEADS, BN, OUT), lambda i: (0, i, 0)),
            pl.BlockSpec((HEADS, NPAD), lambda i: (0, 0)),
            pl.BlockSpec((HEADS, OUT), lambda i: (0, 0)),
        ],
        out_specs=pl.BlockSpec((BN, HID), lambda i: (i, 0)),
        out_shape=jax.ShapeDtypeStruct((NPAD, HID), jnp.float32),
    )(num8, den8, b)


def _final(num8, den8, b, lin_w, lin_b, batch3):
    BN = 1024
    nsteps = NPAD // BN

    def body(n_ref, d_ref, b_ref, lw_ref, lb_ref, bt_ref, o_ref, sums, cnt):
        i = pl.program_id(0)

        @pl.when(i == 0)
        def _():
            sums[...] = jnp.zeros_like(sums)
            cnt[...] = jnp.zeros_like(cnt)

        z = jnp.zeros((BN, ACT), jnp.float32)
        for h in range(HEADS):
            den = d_ref[h, pl.ds(i * BN, BN)][:, None] + 1e-16
            v = n_ref[h] / den + b_ref[h][None, :]
            v = jnp.maximum(v, 0.0)
            z = z + jnp.dot(v, lw_ref[h * OUT:(h + 1) * OUT, :],
                            preferred_element_type=jnp.float32)
        z = jnp.tanh(z + lb_ref[0][None, :])
        bt = bt_ref[0, 0, :]
        onehot = (bt[:, None] == lax.iota(jnp.int32, G)[None, :]).astype(jnp.float32)
        sums[...] += lax.dot_general(onehot, z, (((0,), (0,)), ((), ())),
                                     preferred_element_type=jnp.float32)
        cnt[...] += lax.dot_general(onehot, jnp.ones((BN, 1), jnp.float32),
                                    (((0,), (0,)), ((), ())),
                                    preferred_element_type=jnp.float32)

        @pl.when(i == nsteps - 1)
        def _():
            o_ref[...] = sums[...] / jnp.maximum(cnt[...], 1.0)

    return pl.pallas_call(
        body,
        grid=(nsteps,),
        in_specs=[
            pl.BlockSpec((HEADS, BN, OUT), lambda i: (0, i, 0)),
            pl.BlockSpec((HEADS, NPAD), lambda i: (0, 0)),
            pl.BlockSpec((HEADS, OUT), lambda i: (0, 0)),
            pl.BlockSpec((HID, ACT), lambda i: (0, 0)),
            pl.BlockSpec((1, ACT), lambda i: (0, 0)),
            pl.BlockSpec((1, 1, BN), lambda i: (i, 0, 0)),
        ],
        out_specs=pl.BlockSpec((G, ACT), lambda i: (0, 0)),
        out_shape=jax.ShapeDtypeStruct((G, ACT), jnp.float32),
        scratch_shapes=[pltpu.VMEM((G, ACT), jnp.float32),
                        pltpu.VMEM((G, 1), jnp.float32)],
    )(num8, den8, b, lin_w, lin_b, batch3)


@functools.partial(
    pl.kernel, mesh=_mesh,
    out_type=(jax.ShapeDtypeStruct((E, L), jnp.float32),
              jax.ShapeDtypeStruct((E,), jnp.int32)),
    scratch_types=[
        pltpu.VMEM((EBLK,), jnp.int32),
        pltpu.VMEM((EBLK,), jnp.int32),
        pltpu.VMEM((EBLK,), jnp.int32),
        pltpu.VMEM((EBLK,), jnp.float32),
        pltpu.VMEM((EBLK,), jnp.int32),
        pltpu.VMEM((EBLK,), jnp.int32),
        pltpu.VMEM((EBLK,), jnp.int32),
        pltpu.VMEM((EBLK, 128), jnp.float32),
        pltpu.VMEM((EBLK, 128), jnp.float32),
        pltpu.VMEM((L,), jnp.float32),
        pltpu.VMEM((EBLK, L), jnp.float32),
        pltpu.SemaphoreType.DMA,
    ],
)
def _pass_a(src_hbm, dst_hbm, et_hbm, ea_hbm, qk_hbm, cc_hbm,
            gE_hbm, sidx8_hbm,
            src_v, dst_v, et_v, ea_v, qidx_v, kidx_v, sidx8_v,
            qrow_v, krow_v, cc_v, g_v, sem):
    c = lax.axis_index("c")
    s = lax.axis_index("s")
    wid = s * NC + c
    nblk = jnp.where(wid < NBLK_TOTAL - (NBLK_TOTAL // NW) * NW,
                     NBLK_TOTAL // NW + 1, NBLK_TOTAL // NW)
    pltpu.sync_copy(cc_hbm.at[0, :], cc_v)
    lane = lax.iota(jnp.int32, L)
    perm_k = (lane & (HEADS - 1)) + HEADS

    def blk(b, _):
        eoff = (b * NW + wid) * EBLK
        pltpu.sync_copy(src_hbm.at[pl.ds(eoff, EBLK)], src_v)
        pltpu.sync_copy(dst_hbm.at[pl.ds(eoff, EBLK)], dst_v)
        pltpu.sync_copy(et_hbm.at[pl.ds(eoff, EBLK)], et_v)
        pltpu.sync_copy(ea_hbm.at[pl.ds(eoff, EBLK)], ea_v)

        def idx_body(i, _):
            sl = pl.ds(i * L, L)
            t16 = et_v[sl]
            qi = t16 * N + dst_v[sl]
            ki = t16 * N + src_v[sl]
            qidx_v[sl] = qi
            kidx_v[sl] = ki
            sidx8_v[sl] = ki * 8
            return 0

        lax.fori_loop(0, EBLK // L, idx_body, 0)
        pltpu.sync_copy(sidx8_v, sidx8_hbm.at[pl.ds(eoff, EBLK)])
        pltpu.async_copy(qk_hbm.at[qidx_v], qrow_v, sem).wait()
        pltpu.async_copy(qk_hbm.at[kidx_v], krow_v, sem).wait()
        ccv = cc_v[...]

        def chunk_body(i, _):
            ea_ch = ea_v[pl.ds(i * L, L)]
            for jj in range(L):
                j = i * L + jj
                qrow = qrow_v[j, pl.ds(0, L)]
                krow = krow_v[j, pl.ds(0, L)]
                kshift = krow.at[perm_k].get(mode='promise_in_bounds')
                eaj = ea_ch.at[jnp.full((L,), jj, jnp.int32)].get(
                    mode='promise_in_bounds')
                a = qrow + kshift + eaj * ccv
                a = jnp.where(a >= 0.0, a, a * 0.2)
                g_v[j] = jnp.exp(a)
            return 0

        lax.fori_loop(0, EBLK // L, chunk_body, 0)
        pltpu.sync_copy(g_v, gE_hbm.at[pl.ds(eoff, EBLK), :])
        return 0

    lax.fori_loop(0, nblk, blk, 0)


_HPC = HEADS // NC
_ROWS = NPAD // NS
_NROWS = 632
_NROWS_LAST = N - (NS - 1) * _NROWS


@functools.partial(
    pl.kernel, mesh=_mesh,
    out_type=(jax.ShapeDtypeStruct((HEADS, NPAD, OUT), jnp.float32),
              jax.ShapeDtypeStruct((HEADS * NPAD,), jnp.float32)),
    scratch_types=[
        pltpu.VMEM((EBLK,), jnp.int32),
        pltpu.VMEM((EBLK,), jnp.int32),
        pltpu.VMEM((EBLK, L), jnp.float32),
        pltpu.VMEM((EBLK,), jnp.float32),
        pltpu.VMEM((EBLK, OUT), jnp.float32),
        pltpu.VMEM((EBLK, OUT), jnp.float32),
        pltpu.VMEM((_ROWS,), jnp.float32),
        pltpu.VMEM_SHARED((N, OUT), jnp.float32),
        pltpu.VMEM_SHARED((NPAD,), jnp.float32),
        pltpu.SemaphoreType.DMA,
    ],
)
def _pass_b(dst_hbm, sidx8_hbm, gE_hbm, xrel8_hbm,
            num8_hbm, den8_hbm,
            dst_v, idxg_v, g_v, dsrc_v, xg_v, z_v, zd_v, accn, accd, sem):
    c = lax.axis_index("c")
    s = lax.axis_index("s")
    nblk = jnp.where(s < NBLK_TOTAL - (NBLK_TOTAL // NS) * NS,
                     NBLK_TOTAL // NS + 1, NBLK_TOTAL // NS)
    rbase = s * _ROWS

    def zrow(i, _):
        for k in range(OUT // L):
            z_v[i, pl.ds(k * L, L)] = jnp.zeros((L,), jnp.float32)
        return 0

    lax.fori_loop(0, EBLK, zrow, 0)

    def zvec(i, _):
        zd_v[pl.ds(i * L, L)] = jnp.zeros((L,), jnp.float32)
        return 0

    lax.fori_loop(0, _ROWS // L, zvec, 0)

    def _zero_num(base, count):
        off = 0
        while off < count:
            step = min(EBLK, count - off)
            pltpu.sync_copy(z_v.at[pl.ds(0, step), :],
                            accn.at[pl.ds(base + off, step), :])
            off += step

    for p in range(_HPC):
        h = c * _HPC + p

        @pl.when(s < NS - 1)
        def _():
            _zero_num(s * _NROWS, _NROWS)

        @pl.when(s == NS - 1)
        def _():
            _zero_num((NS - 1) * _NROWS, _NROWS_LAST)

        pltpu.sync_copy(zd_v, accd.at[pl.ds(rbase, _ROWS)])
        plsc.subcore_barrier()

        def blk(b, _):
            eoff = (b * NS + s) * EBLK
            pltpu.sync_copy(dst_hbm.at[pl.ds(eoff, EBLK)], dst_v)
            pltpu.sync_copy(sidx8_hbm.at[pl.ds(eoff, EBLK)], idxg_v)

            def addh(i, _):
                sl = pl.ds(i * L, L)
                idxg_v[sl] = idxg_v[sl] + h
                return 0

            lax.fori_loop(0, EBLK // L, addh, 0)
            pltpu.sync_copy(gE_hbm.at[pl.ds(eoff, EBLK), :], g_v)
            pltpu.async_copy(xrel8_hbm.at[idxg_v], xg_v, sem).wait()
            lane = lax.iota(jnp.int32, L)
            hidx = jnp.full((L,), h, jnp.int32)

            def scale(i, _):
                dv = jnp.zeros((L,), jnp.float32)
                for jj in range(L):
                    j = i * L + jj
                    gj = g_v[j].at[hidx].get(mode='promise_in_bounds')
                    dv = jnp.where(lane == jj, gj, dv)
                    for k in range(OUT // L):
                        sl = pl.ds(k * L, L)
                        xg_v[j, sl] = xg_v[j, sl] * gj
                dsrc_v[pl.ds(i * L, L)] = dv
                return 0

            lax.fori_loop(0, EBLK // L, scale, 0)
            pltpu.sync_copy(xg_v, accn.at[dst_v], add=True)
            pltpu.sync_copy(dsrc_v, accd.at[dst_v], add=True)
            return 0

        lax.fori_loop(0, nblk, blk, 0)
        plsc.subcore_barrier()

        @pl.when(s < NS - 1)
        def _():
            nb = s * _NROWS
            pltpu.sync_copy(accn.at[pl.ds(nb, _NROWS), :],
                            num8_hbm.at[h, pl.ds(nb, _NROWS), :])

        @pl.when(s == NS - 1)
        def _():
            nb = (NS - 1) * _NROWS
            pltpu.sync_copy(accn.at[pl.ds(nb, _NROWS_LAST), :],
                            num8_hbm.at[h, pl.ds(nb, _NROWS_LAST), :])
            pltpu.sync_copy(z_v.at[pl.ds(0, EBLK), :],
                            num8_hbm.at[h, pl.ds(N, EBLK), :])
            pltpu.sync_copy(z_v.at[pl.ds(0, NPAD - N - EBLK), :],
                            num8_hbm.at[h, pl.ds(N + EBLK, NPAD - N - EBLK), :])

        pltpu.sync_copy(accd.at[pl.ds(rbase, _ROWS)],
                        den8_hbm.at[pl.ds(h * NPAD + rbase, _ROWS)])
        plsc.subcore_barrier()


def _layer(x, src, dst, et, ea, W, q, k, We, e, b):
    xrel = _mm_xrel(x, W)
    qk_cat = jnp.concatenate([q, k], axis=1)
    Wqk = _mm_wqk(W, qk_cat)
    qktab = _mm_qktab(x, Wqk).reshape(R * N, 128)
    cc = _cc_vec(We, e)
    gT, sidx8 = _pass_a(src, dst, et, ea, qktab, cc)
    xrel8 = xrel.reshape(R * N * HEADS, OUT)
    num8, den8p = _pass_b(dst, sidx8, gT, xrel8)
    return num8, den8p.reshape(HEADS, NPAD)


def kernel(x, edge_index, edge_type, edge_attr, batch,
           W1, q1, k1, We1, e1, b1,
           W2, q2, k2, We2, e2, b2,
           W3, q3, k3, We3, e3, b3,
           lin_w, lin_b):
    src = edge_index[0]
    dst = edge_index[1]
    ea = edge_attr[:, 0]
    batch3 = jnp.pad(batch, (0, NPAD - N),
                     constant_values=G).reshape(NPAD // 1024, 1, 1024)

    num8, den8 = _layer(x, src, dst, edge_type, ea, W1, q1, k1, We1, e1, b1)
    h = _normalize(num8, den8, b1.reshape(HEADS, OUT))[:N]
    num8, den8 = _layer(h, src, dst, edge_type, ea, W2, q2, k2, We2, e2, b2)
    h = _normalize(num8, den8, b2.reshape(HEADS, OUT))[:N]
    num8, den8 = _layer(h, src, dst, edge_type, ea, W3, q3, k3, We3, e3, b3)
    return _final(num8, den8, b3.reshape(HEADS, OUT), lin_w,
                  lin_b.reshape(1, ACT), batch3)


# speedup vs baseline: 19.9965x; 1.7591x over previous
import functools

import jax
import jax.numpy as jnp
from jax import lax
from jax.experimental import pallas as pl
from jax.experimental.pallas import tpu as pltpu
from jax.experimental.pallas import tpu_sc as plsc

N = 10000
E = 160000
D = 128
HEADS = 8
OUT = 128
R = 3
ACT = 16
G = 16
HID = HEADS * OUT

NC, NS, L = 2, 16, 16
NW = NC * NS
EBLK = 128
NBLK_TOTAL = E // EBLK
NPAD = 10240

_mesh = plsc.VectorSubcoreMesh(core_axis_name="c", subcore_axis_name="s",
                               num_cores=NC, num_subcores=NS)


def _mm_xrel(x, W):
    IN = x.shape[1]
    BN, BH = 1000, 512

    def body(x_ref, w_ref, o_ref):
        o_ref[...] = jnp.dot(x_ref[...], w_ref[0],
                             preferred_element_type=jnp.float32)[None]

    return pl.pallas_call(
        body,
        grid=(R, N // BN, HID // BH),
        in_specs=[
            pl.BlockSpec((BN, IN), lambda r, i, j: (i, 0)),
            pl.BlockSpec((1, IN, BH), lambda r, i, j: (r, 0, j)),
        ],
        out_specs=pl.BlockSpec((1, BN, BH), lambda r, i, j: (r, i, j)),
        out_shape=jax.ShapeDtypeStruct((R, N, HID), jnp.float32),
    )(x, W)


def _mm_wqk(W, qk):
    IN = W.shape[1]

    def body(w_ref, qk_ref, o_ref):
        o_ref[...] = jnp.dot(w_ref[0], qk_ref[...],
                             preferred_element_type=jnp.float32)[None]

    return pl.pallas_call(
        body,
        grid=(R,),
        in_specs=[
            pl.BlockSpec((1, IN, HID), lambda r: (r, 0, 0)),
            pl.BlockSpec((HID, 2 * HEADS), lambda r: (0, 0)),
        ],
        out_specs=pl.BlockSpec((1, IN, 2 * HEADS), lambda r: (r, 0, 0)),
        out_shape=jax.ShapeDtypeStruct((R, IN, 2 * HEADS), jnp.float32),
    )(W, qk)


def _mm_qktab(x, Wqk):
    IN = x.shape[1]
    BN = 1000

    def body(x_ref, w_ref, o_ref):
        v = jnp.dot(x_ref[...], w_ref[0],
                    preferred_element_type=jnp.float32)
        o_ref[...] = jnp.concatenate(
            [v, jnp.zeros((BN, 128 - 2 * HEADS), jnp.float32)], axis=1)[None]

    return pl.pallas_call(
        body,
        grid=(R, N // BN),
        in_specs=[
            pl.BlockSpec((BN, IN), lambda r, i: (i, 0)),
            pl.BlockSpec((1, IN, 2 * HEADS), lambda r, i: (r, 0, 0)),
        ],
        out_specs=pl.BlockSpec((1, BN, 128), lambda r, i: (r, i, 0)),
        out_shape=jax.ShapeDtypeStruct((R, N, 128), jnp.float32),
    )(x, Wqk)


def _cc_vec(We, e):
    def body(we_ref, e_ref, o_ref):
        c = jnp.dot(we_ref[...], e_ref[...],
                    preferred_element_type=jnp.float32)
        o_ref[:, 0:HEADS] = c
        o_ref[:, HEADS:2 * HEADS] = c

    return pl.pallas_call(
        body,
        out_shape=jax.ShapeDtypeStruct((1, 2 * HEADS), jnp.float32),
    )(We, e)


def _normalize(num8, den8, b):
    BN = 1024

    def body(n_ref, d_ref, b_ref, o_ref):
        i = pl.program_id(0)
        for h in range(HEADS):
            den = d_ref[h, pl.ds(i * BN, BN)][:, None] + 1e-16
            v = n_ref[h] / den + b_ref[h][None, :]
            o_ref[:, h * OUT:(h + 1) * OUT] = jnp.maximum(v, 0.0)

    return pl.pallas_call(
        body,
        grid=(NPAD // BN,),
        in_specs=[
            pl.BlockSpec((HEADS, BN, OUT), lambda i: (0, i, 0)),
            pl.BlockSpec((HEADS, NPAD), lambda i: (0, 0)),
            pl.BlockSpec((HEADS, OUT), lambda i: (0, 0)),
        ],
        out_specs=pl.BlockSpec((BN, HID), lambda i: (i, 0)),
        out_shape=jax.ShapeDtypeStruct((NPAD, HID), jnp.float32),
    )(num8, den8, b)


def _final(num8, den8, b, lin_w, lin_b, batch3):
    BN = 1024
    nsteps = NPAD // BN

    def body(n_ref, d_ref, b_ref, lw_ref, lb_ref, bt_ref, o_ref, sums, cnt):
        i = pl.program_id(0)

        @pl.when(i == 0)
        def _():
            sums[...] = jnp.zeros_like(sums)
            cnt[...] = jnp.zeros_like(cnt)

        z = jnp.zeros((BN, ACT), jnp.float32)
        for h in range(HEADS):
            den = d_ref[h, pl.ds(i * BN, BN)][:, None] + 1e-16
            v = n_ref[h] / den + b_ref[h][None, :]
            v = jnp.maximum(v, 0.0)
            z = z + jnp.dot(v, lw_ref[h * OUT:(h + 1) * OUT, :],
                            preferred_element_type=jnp.float32)
        z = jnp.tanh(z + lb_ref[0][None, :])
        bt = bt_ref[0, 0, :]
        onehot = (bt[:, None] == lax.iota(jnp.int32, G)[None, :]).astype(jnp.float32)
        sums[...] += lax.dot_general(onehot, z, (((0,), (0,)), ((), ())),
                                     preferred_element_type=jnp.float32)
        cnt[...] += lax.dot_general(onehot, jnp.ones((BN, 1), jnp.float32),
                                    (((0,), (0,)), ((), ())),
                                    preferred_element_type=jnp.float32)

        @pl.when(i == nsteps - 1)
        def _():
            o_ref[...] = sums[...] / jnp.maximum(cnt[...], 1.0)

    return pl.pallas_call(
        body,
        grid=(nsteps,),
        in_specs=[
            pl.BlockSpec((HEADS, BN, OUT), lambda i: (0, i, 0)),
            pl.BlockSpec((HEADS, NPAD), lambda i: (0, 0)),
            pl.BlockSpec((HEADS, OUT), lambda i: (0, 0)),
            pl.BlockSpec((HID, ACT), lambda i: (0, 0)),
            pl.BlockSpec((1, ACT), lambda i: (0, 0)),
            pl.BlockSpec((1, 1, BN), lambda i: (i, 0, 0)),
        ],
        out_specs=pl.BlockSpec((G, ACT), lambda i: (0, 0)),
        out_shape=jax.ShapeDtypeStruct((G, ACT), jnp.float32),
        scratch_shapes=[pltpu.VMEM((G, ACT), jnp.float32),
                        pltpu.VMEM((G, 1), jnp.float32)],
    )(num8, den8, b, lin_w, lin_b, batch3)


@functools.partial(
    pl.kernel, mesh=_mesh,
    out_type=(jax.ShapeDtypeStruct((E, L), jnp.float32),
              jax.ShapeDtypeStruct((E,), jnp.int32)),
    scratch_types=[
        pltpu.VMEM((EBLK,), jnp.int32),
        pltpu.VMEM((EBLK,), jnp.int32),
        pltpu.VMEM((EBLK,), jnp.int32),
        pltpu.VMEM((EBLK,), jnp.float32),
        pltpu.VMEM((EBLK,), jnp.int32),
        pltpu.VMEM((EBLK,), jnp.int32),
        pltpu.VMEM((EBLK,), jnp.int32),
        pltpu.VMEM((EBLK, 128), jnp.float32),
        pltpu.VMEM((EBLK, 128), jnp.float32),
        pltpu.VMEM((L,), jnp.float32),
        pltpu.VMEM((EBLK, L), jnp.float32),
        pltpu.SemaphoreType.DMA,
    ],
)
def _pass_a(src_hbm, dst_hbm, et_hbm, ea_hbm, qk_hbm, cc_hbm,
            gE_hbm, sidx8_hbm,
            src_v, dst_v, et_v, ea_v, qidx_v, kidx_v, sidx8_v,
            qrow_v, krow_v, cc_v, g_v, sem):
    c = lax.axis_index("c")
    s = lax.axis_index("s")
    wid = s * NC + c
    nblk = jnp.where(wid < NBLK_TOTAL - (NBLK_TOTAL // NW) * NW,
                     NBLK_TOTAL // NW + 1, NBLK_TOTAL // NW)
    pltpu.sync_copy(cc_hbm.at[0, :], cc_v)
    lane = lax.iota(jnp.int32, L)
    perm_k = (lane & (HEADS - 1)) + HEADS

    def blk(b, _):
        eoff = (b * NW + wid) * EBLK
        pltpu.sync_copy(src_hbm.at[pl.ds(eoff, EBLK)], src_v)
        pltpu.sync_copy(dst_hbm.at[pl.ds(eoff, EBLK)], dst_v)
        pltpu.sync_copy(et_hbm.at[pl.ds(eoff, EBLK)], et_v)
        pltpu.sync_copy(ea_hbm.at[pl.ds(eoff, EBLK)], ea_v)

        def idx_body(i, _):
            sl = pl.ds(i * L, L)
            t16 = et_v[sl]
            qi = t16 * N + dst_v[sl]
            ki = t16 * N + src_v[sl]
            qidx_v[sl] = qi
            kidx_v[sl] = ki
            sidx8_v[sl] = ki * 8
            return 0

        lax.fori_loop(0, EBLK // L, idx_body, 0)
        pltpu.sync_copy(sidx8_v, sidx8_hbm.at[pl.ds(eoff, EBLK)])
        pltpu.async_copy(qk_hbm.at[qidx_v], qrow_v, sem).wait()
        pltpu.async_copy(qk_hbm.at[kidx_v], krow_v, sem).wait()
        ccv = cc_v[...]

        def chunk_body(i, _):
            ea_ch = ea_v[pl.ds(i * L, L)]
            for jj in range(L):
                j = i * L + jj
                qrow = qrow_v[j, pl.ds(0, L)]
                krow = krow_v[j, pl.ds(0, L)]
                kshift = krow.at[perm_k].get(mode='promise_in_bounds')
                eaj = ea_ch.at[jnp.full((L,), jj, jnp.int32)].get(
                    mode='promise_in_bounds')
                a = qrow + kshift + eaj * ccv
                a = jnp.where(a >= 0.0, a, a * 0.2)
                g_v[j] = jnp.exp(a)
            return 0

        lax.fori_loop(0, EBLK // L, chunk_body, 0)
        pltpu.sync_copy(g_v, gE_hbm.at[pl.ds(eoff, EBLK), :])
        return 0

    lax.fori_loop(0, nblk, blk, 0)


_HPC = HEADS // NC
_ROWS = NPAD // NS
_NROWS = 632
_NROWS_LAST = N - (NS - 1) * _NROWS


_BMAX = NBLK_TOTAL // NS + 2
_UNR = 6


@functools.partial(
    pl.kernel, mesh=_mesh,
    out_type=(jax.ShapeDtypeStruct((HEADS, NPAD, OUT), jnp.float32),
              jax.ShapeDtypeStruct((HEADS * NPAD,), jnp.float32)),
    scratch_types=[
        pltpu.VMEM((EBLK,), jnp.int32),
        pltpu.VMEM((EBLK,), jnp.int32),
        pltpu.VMEM((EBLK,), jnp.int32),
        pltpu.VMEM((EBLK,), jnp.int32),
        pltpu.VMEM((EBLK,), jnp.int32),
        pltpu.VMEM((EBLK,), jnp.int32),
        pltpu.VMEM((EBLK,), jnp.int32),
        pltpu.VMEM((EBLK,), jnp.int32),
        pltpu.VMEM((EBLK, L), jnp.float32),
        pltpu.VMEM((EBLK,), jnp.float32),
        pltpu.VMEM((EBLK,), jnp.float32),
        pltpu.VMEM((EBLK, OUT), jnp.float32),
        pltpu.VMEM((EBLK, OUT), jnp.float32),
        pltpu.VMEM_SHARED((N, OUT), jnp.float32),
        pltpu.VMEM_SHARED((NPAD,), jnp.float32),
        pltpu.SemaphoreType.DMA,
        pltpu.SemaphoreType.DMA,
        pltpu.SemaphoreType.DMA,
        pltpu.SemaphoreType.DMA,
        pltpu.SemaphoreType.DMA,
        pltpu.SemaphoreType.DMA,
        pltpu.SemaphoreType.DMA,
        pltpu.SemaphoreType.DMA,
    ],
)
def _pass_b(dst2_hbm, sidx2_hbm, gE_hbm, xrel8_hbm,
            num8_hbm, den8_hbm,
            dm0, dm1, dm2, sm0, sm1, sm2, ix0, ix1,
            g_v, d0_v, d1_v, x0_v, x1_v,
            accn, accd,
            smm0, smm1, smm2, smg0, smg1, sml, smsc0, smsc1):
    c = lax.axis_index("c")
    s = lax.axis_index("s")
    nblk = jnp.where(s < NBLK_TOTAL - (NBLK_TOTAL // NS) * NS,
                     NBLK_TOTAL // NS + 1, NBLK_TOTAL // NS)
    rbase = s * _ROWS
    lane = lax.iota(jnp.int32, L)

    DM = (dm0, dm1, dm2)
    SM = (sm0, sm1, sm2)
    IX = (ix0, ix1)
    D_ = (d0_v, d1_v)
    X_ = (x0_v, x1_v)
    SMM = (smm0, smm1, smm2)
    SMG = (smg0, smg1)
    SSC = (smsc0, smsc1)

    def issue_meta(b, m):
        row = b * NS + s
        pltpu.async_copy(dst2_hbm.at[row], DM[m], SMM[m])
        pltpu.async_copy(sidx2_hbm.at[row], SM[m], SMM[m])

    def wait_meta(m):
        pltpu.make_async_copy(dst2_hbm.at[0], DM[m], SMM[m]).wait()
        pltpu.make_async_copy(sidx2_hbm.at[0], SM[m], SMM[m]).wait()

    def issue_gather(xs, i):
        pltpu.async_copy(xrel8_hbm.at[IX[i]], X_[xs], SMG[xs])

    def wait_gather(xs, i):
        pltpu.make_async_copy(xrel8_hbm.at[IX[i]], X_[xs], SMG[xs]).wait()

    def issue_g(b):
        eoff = (b * NS + s) * EBLK
        pltpu.async_copy(gE_hbm.at[pl.ds(eoff, EBLK), :], g_v, sml)

    def wait_g():
        pltpu.make_async_copy(gE_hbm.at[pl.ds(0, EBLK), :], g_v, sml).wait()

    def issue_sc(xs, m):
        pltpu.async_copy(X_[xs], accn.at[DM[m]], SSC[xs], add=True)
        pltpu.async_copy(D_[xs], accd.at[DM[m]], SSC[xs], add=True)

    def wait_sc(xs):
        pltpu.make_async_copy(X_[xs], accn.at[DM[0]], SSC[xs]).wait()
        pltpu.make_async_copy(D_[xs], accd.at[DM[0]], SSC[xs]).wait()

    def comp_idx(i, m, h):
        for k in range(EBLK // L):
            sl = pl.ds(k * L, L)
            IX[i][sl] = SM[m][sl] + h

    def zero_x0():
        def zr(i, _):
            for k in range(OUT // L):
                x0_v[i, pl.ds(k * L, L)] = jnp.zeros((L,), jnp.float32)
            return 0
        lax.fori_loop(0, EBLK, zr, 0)

    def phase(p, _):
        h = c * _HPC + p
        hidx = jnp.full((L,), h, jnp.int32)

        zero_x0()
        for k in range(EBLK // L):
            d0_v[pl.ds(k * L, L)] = jnp.zeros((L,), jnp.float32)

        @pl.when(s < NS - 1)
        def _():
            nb = s * _NROWS
            for k in range(_NROWS // EBLK):
                pltpu.sync_copy(x0_v, accn.at[pl.ds(nb + k * EBLK, EBLK), :])
            pltpu.sync_copy(x0_v.at[pl.ds(0, _NROWS % EBLK), :],
                            accn.at[pl.ds(nb + (_NROWS // EBLK) * EBLK,
                                          _NROWS % EBLK), :])

        @pl.when(s == NS - 1)
        def _():
            nb = (NS - 1) * _NROWS
            for k in range(_NROWS_LAST // EBLK):
                pltpu.sync_copy(x0_v, accn.at[pl.ds(nb + k * EBLK, EBLK), :])
            pltpu.sync_copy(x0_v.at[pl.ds(0, _NROWS_LAST % EBLK), :],
                            accn.at[pl.ds(nb + (_NROWS_LAST // EBLK) * EBLK,
                                          _NROWS_LAST % EBLK), :])

        for k in range(_ROWS // EBLK):
            pltpu.sync_copy(d0_v, accd.at[pl.ds(rbase + k * EBLK, EBLK)])
        plsc.subcore_barrier()

        issue_meta(0, 0)
        issue_meta(1, 1)
        issue_g(0)
        wait_meta(0)
        comp_idx(0, 0, h)
        issue_gather(0, 0)

        def six(b6, _):
            for u in range(_UNR):
                b = b6 * _UNR + u
                xs = u % 2
                m = u % 3

                @pl.when(b < nblk)
                def _():
                    @pl.when(b >= 1)
                    def _():
                        wait_sc(xs ^ 1)

                    @pl.when(b + 1 < nblk)
                    def _():
                        wait_meta((u + 1) % 3)
                        comp_idx((u + 1) % 2, (u + 1) % 3, h)
                        issue_gather(xs ^ 1, (u + 1) % 2)

                    @pl.when(b + 2 < nblk)
                    def _():
                        issue_meta(b + 2, (u + 2) % 3)

                    wait_gather(xs, u % 2)
                    wait_g()

                    def chunk(i, _):
                        dv = jnp.zeros((L,), jnp.float32)
                        for jj in range(L):
                            j = i * L + jj
                            gj = g_v[j].at[hidx].get(mode='promise_in_bounds')
                            dv = jnp.where(lane == jj, gj, dv)
                            for kk in range(OUT // L):
                                sl = pl.ds(kk * L, L)
                                X_[xs][j, sl] = X_[xs][j, sl] * gj
                        D_[xs][pl.ds(i * L, L)] = dv
                        return 0

                    lax.fori_loop(0, EBLK // L, chunk, 0)
                    issue_sc(xs, m)

                    @pl.when(b + 1 < nblk)
                    def _():
                        issue_g(b + 1)
            return 0

        lax.fori_loop(0, (_BMAX + _UNR - 1) // _UNR, six, 0)

        @pl.when(nblk % 2 == 1)
        def _():
            wait_sc(0)

        @pl.when(nblk % 2 == 0)
        def _():
            wait_sc(1)

        plsc.subcore_barrier()

        @pl.when(s < NS - 1)
        def _():
            nb = s * _NROWS
            pltpu.sync_copy(accn.at[pl.ds(nb, _NROWS), :],
                            num8_hbm.at[h, pl.ds(nb, _NROWS), :])

        @pl.when(s == NS - 1)
        def _():
            nb = (NS - 1) * _NROWS
            pltpu.sync_copy(accn.at[pl.ds(nb, _NROWS_LAST), :],
                            num8_hbm.at[h, pl.ds(nb, _NROWS_LAST), :])
            zero_x0()
            pltpu.sync_copy(x0_v, num8_hbm.at[h, pl.ds(N, EBLK), :])
            pltpu.sync_copy(x0_v.at[pl.ds(0, NPAD - N - EBLK), :],
                            num8_hbm.at[h, pl.ds(N + EBLK, NPAD - N - EBLK), :])

        pltpu.sync_copy(accd.at[pl.ds(rbase, _ROWS)],
                        den8_hbm.at[pl.ds(h * NPAD + rbase, _ROWS)])
        plsc.subcore_barrier()
        return 0

    lax.fori_loop(0, _HPC, phase, 0)


def _layer(x, src, dst, et, ea, W, q, k, We, e, b):
    xrel = _mm_xrel(x, W)
    qk_cat = jnp.concatenate([q, k], axis=1)
    Wqk = _mm_wqk(W, qk_cat)
    qktab = _mm_qktab(x, Wqk).reshape(R * N, 128)
    cc = _cc_vec(We, e)
    gT, sidx8 = _pass_a(src, dst, et, ea, qktab, cc)
    xrel8 = xrel.reshape(R * N * HEADS, OUT)
    num8, den8p = _pass_b(dst.reshape(NBLK_TOTAL, EBLK),
                          sidx8.reshape(NBLK_TOTAL, EBLK), gT, xrel8)
    return num8, den8p.reshape(HEADS, NPAD)


def kernel(x, edge_index, edge_type, edge_attr, batch,
           W1, q1, k1, We1, e1, b1,
           W2, q2, k2, We2, e2, b2,
           W3, q3, k3, We3, e3, b3,
           lin_w, lin_b):
    src = edge_index[0]
    dst = edge_index[1]
    ea = edge_attr[:, 0]
    batch3 = jnp.pad(batch, (0, NPAD - N),
                     constant_values=G).reshape(NPAD // 1024, 1, 1024)

    num8, den8 = _layer(x, src, dst, edge_type, ea, W1, q1, k1, We1, e1, b1)
    h = _normalize(num8, den8, b1.reshape(HEADS, OUT))[:N]
    num8, den8 = _layer(h, src, dst, edge_type, ea, W2, q2, k2, We2, e2, b2)
    h = _normalize(num8, den8, b2.reshape(HEADS, OUT))[:N]
    num8, den8 = _layer(h, src, dst, edge_type, ea, W3, q3, k3, We3, e3, b3)
    return _final(num8, den8, b3.reshape(HEADS, OUT), lin_w,
                  lin_b.reshape(1, ACT), batch3)


# speedup vs baseline: 20.3251x; 1.0164x over previous
import functools

import jax
import jax.numpy as jnp
from jax import lax
from jax.experimental import pallas as pl
from jax.experimental.pallas import tpu as pltpu
from jax.experimental.pallas import tpu_sc as plsc

N = 10000
E = 160000
D = 128
HEADS = 8
OUT = 128
R = 3
ACT = 16
G = 16
HID = HEADS * OUT

NC, NS, L = 2, 16, 16
NW = NC * NS
EBLK = 128
NBLK_TOTAL = E // EBLK
NPAD = 10240

_mesh = plsc.VectorSubcoreMesh(core_axis_name="c", subcore_axis_name="s",
                               num_cores=NC, num_subcores=NS)


def _mm_xrel(x, W):
    IN = x.shape[1]
    BN, BH = 1000, 512

    def body(x_ref, w_ref, o_ref):
        o_ref[...] = jnp.dot(x_ref[...], w_ref[0],
                             preferred_element_type=jnp.float32)[None]

    return pl.pallas_call(
        body,
        grid=(R, N // BN, HID // BH),
        in_specs=[
            pl.BlockSpec((BN, IN), lambda r, i, j: (i, 0)),
            pl.BlockSpec((1, IN, BH), lambda r, i, j: (r, 0, j)),
        ],
        out_specs=pl.BlockSpec((1, BN, BH), lambda r, i, j: (r, i, j)),
        out_shape=jax.ShapeDtypeStruct((R, N, HID), jnp.float32),
    )(x, W)


def _mm_wqk(W, qk):
    IN = W.shape[1]

    def body(w_ref, qk_ref, o_ref):
        o_ref[...] = jnp.dot(w_ref[0], qk_ref[...],
                             preferred_element_type=jnp.float32)[None]

    return pl.pallas_call(
        body,
        grid=(R,),
        in_specs=[
            pl.BlockSpec((1, IN, HID), lambda r: (r, 0, 0)),
            pl.BlockSpec((HID, 2 * HEADS), lambda r: (0, 0)),
        ],
        out_specs=pl.BlockSpec((1, IN, 2 * HEADS), lambda r: (r, 0, 0)),
        out_shape=jax.ShapeDtypeStruct((R, IN, 2 * HEADS), jnp.float32),
    )(W, qk)


def _mm_qktab(x, Wqk):
    IN = x.shape[1]
    BN = 1000

    def body(x_ref, w_ref, o_ref):
        v = jnp.dot(x_ref[...], w_ref[0],
                    preferred_element_type=jnp.float32)
        o_ref[...] = jnp.concatenate(
            [v, jnp.zeros((BN, 128 - 2 * HEADS), jnp.float32)], axis=1)[None]

    return pl.pallas_call(
        body,
        grid=(R, N // BN),
        in_specs=[
            pl.BlockSpec((BN, IN), lambda r, i: (i, 0)),
            pl.BlockSpec((1, IN, 2 * HEADS), lambda r, i: (r, 0, 0)),
        ],
        out_specs=pl.BlockSpec((1, BN, 128), lambda r, i: (r, i, 0)),
        out_shape=jax.ShapeDtypeStruct((R, N, 128), jnp.float32),
    )(x, Wqk)


def _cc_vec(We, e):
    def body(we_ref, e_ref, o_ref):
        c = jnp.dot(we_ref[...], e_ref[...],
                    preferred_element_type=jnp.float32)
        o_ref[:, 0:HEADS] = c
        o_ref[:, HEADS:2 * HEADS] = c

    return pl.pallas_call(
        body,
        out_shape=jax.ShapeDtypeStruct((1, 2 * HEADS), jnp.float32),
    )(We, e)


def _normalize(num8, den8, b):
    BN = 1024

    def body(n_ref, d_ref, b_ref, o_ref):
        i = pl.program_id(0)
        for h in range(HEADS):
            den = d_ref[h, pl.ds(i * BN, BN)][:, None] + 1e-16
            v = n_ref[h] / den + b_ref[h][None, :]
            o_ref[:, h * OUT:(h + 1) * OUT] = jnp.maximum(v, 0.0)

    return pl.pallas_call(
        body,
        grid=(NPAD // BN,),
        in_specs=[
            pl.BlockSpec((HEADS, BN, OUT), lambda i: (0, i, 0)),
            pl.BlockSpec((HEADS, NPAD), lambda i: (0, 0)),
            pl.BlockSpec((HEADS, OUT), lambda i: (0, 0)),
        ],
        out_specs=pl.BlockSpec((BN, HID), lambda i: (i, 0)),
        out_shape=jax.ShapeDtypeStruct((NPAD, HID), jnp.float32),
    )(num8, den8, b)


def _final(num8, den8, b, lin_w, lin_b, batch3):
    BN = 1024
    nsteps = NPAD // BN

    def body(n_ref, d_ref, b_ref, lw_ref, lb_ref, bt_ref, o_ref, sums, cnt):
        i = pl.program_id(0)

        @pl.when(i == 0)
        def _():
            sums[...] = jnp.zeros_like(sums)
            cnt[...] = jnp.zeros_like(cnt)

        z = jnp.zeros((BN, ACT), jnp.float32)
        for h in range(HEADS):
            den = d_ref[h, pl.ds(i * BN, BN)][:, None] + 1e-16
            v = n_ref[h] / den + b_ref[h][None, :]
            v = jnp.maximum(v, 0.0)
            z = z + jnp.dot(v, lw_ref[h * OUT:(h + 1) * OUT, :],
                            preferred_element_type=jnp.float32)
        z = jnp.tanh(z + lb_ref[0][None, :])
        bt = bt_ref[0, 0, :]
        onehot = (bt[:, None] == lax.iota(jnp.int32, G)[None, :]).astype(jnp.float32)
        sums[...] += lax.dot_general(onehot, z, (((0,), (0,)), ((), ())),
                                     preferred_element_type=jnp.float32)
        cnt[...] += lax.dot_general(onehot, jnp.ones((BN, 1), jnp.float32),
                                    (((0,), (0,)), ((), ())),
                                    preferred_element_type=jnp.float32)

        @pl.when(i == nsteps - 1)
        def _():
            o_ref[...] = sums[...] / jnp.maximum(cnt[...], 1.0)

    return pl.pallas_call(
        body,
        grid=(nsteps,),
        in_specs=[
            pl.BlockSpec((HEADS, BN, OUT), lambda i: (0, i, 0)),
            pl.BlockSpec((HEADS, NPAD), lambda i: (0, 0)),
            pl.BlockSpec((HEADS, OUT), lambda i: (0, 0)),
            pl.BlockSpec((HID, ACT), lambda i: (0, 0)),
            pl.BlockSpec((1, ACT), lambda i: (0, 0)),
            pl.BlockSpec((1, 1, BN), lambda i: (i, 0, 0)),
        ],
        out_specs=pl.BlockSpec((G, ACT), lambda i: (0, 0)),
        out_shape=jax.ShapeDtypeStruct((G, ACT), jnp.float32),
        scratch_shapes=[pltpu.VMEM((G, ACT), jnp.float32),
                        pltpu.VMEM((G, 1), jnp.float32)],
    )(num8, den8, b, lin_w, lin_b, batch3)


@functools.partial(
    pl.kernel, mesh=_mesh,
    out_type=(jax.ShapeDtypeStruct((E, L), jnp.float32),
              jax.ShapeDtypeStruct((E,), jnp.int32)),
    scratch_types=[
        pltpu.VMEM((EBLK,), jnp.int32),
        pltpu.VMEM((EBLK,), jnp.int32),
        pltpu.VMEM((EBLK,), jnp.int32),
        pltpu.VMEM((EBLK,), jnp.float32),
        pltpu.VMEM((EBLK,), jnp.int32),
        pltpu.VMEM((EBLK,), jnp.int32),
        pltpu.VMEM((EBLK,), jnp.int32),
        pltpu.VMEM((EBLK, 128), jnp.float32),
        pltpu.VMEM((EBLK, 128), jnp.float32),
        pltpu.VMEM((L,), jnp.float32),
        pltpu.VMEM((EBLK, L), jnp.float32),
        pltpu.SemaphoreType.DMA,
        pltpu.SemaphoreType.DMA,
        pltpu.SemaphoreType.DMA,
    ],
)
def _pass_a(src_hbm, dst_hbm, et_hbm, ea_hbm, qk_hbm, cc_hbm,
            gE_hbm, sidx8_hbm,
            src_v, dst_v, et_v, ea_v, qidx_v, kidx_v, sidx8_v,
            qrow_v, krow_v, cc_v, g_v, sem, sem2, sem3):
    c = lax.axis_index("c")
    s = lax.axis_index("s")
    wid = s * NC + c
    nblk = jnp.where(wid < NBLK_TOTAL - (NBLK_TOTAL // NW) * NW,
                     NBLK_TOTAL // NW + 1, NBLK_TOTAL // NW)
    pltpu.sync_copy(cc_hbm.at[0, :], cc_v)
    lane = lax.iota(jnp.int32, L)
    perm_k = (lane & (HEADS - 1)) + HEADS

    def blk(b, _):
        eoff = (b * NW + wid) * EBLK
        pltpu.sync_copy(src_hbm.at[pl.ds(eoff, EBLK)], src_v)
        pltpu.sync_copy(dst_hbm.at[pl.ds(eoff, EBLK)], dst_v)
        pltpu.sync_copy(et_hbm.at[pl.ds(eoff, EBLK)], et_v)
        pltpu.sync_copy(ea_hbm.at[pl.ds(eoff, EBLK)], ea_v)

        def idx_body(i, _):
            sl = pl.ds(i * L, L)
            t16 = et_v[sl]
            qi = t16 * N + dst_v[sl]
            ki = t16 * N + src_v[sl]
            qidx_v[sl] = qi
            kidx_v[sl] = ki
            sidx8_v[sl] = ki * 8
            return 0

        lax.fori_loop(0, EBLK // L, idx_body, 0)
        pltpu.sync_copy(sidx8_v, sidx8_hbm.at[pl.ds(eoff, EBLK)])
        q_d = pltpu.async_copy(qk_hbm.at[qidx_v], qrow_v, sem)
        k_d = pltpu.async_copy(qk_hbm.at[kidx_v], krow_v, sem2)
        q_d.wait()
        k_d.wait()

        @pl.when(b >= 1)
        def _():
            pltpu.make_async_copy(g_v, gE_hbm.at[pl.ds(0, EBLK), :],
                                  sem3).wait()

        ccv = cc_v[...]

        def chunk_body(i, _):
            ea_ch = ea_v[pl.ds(i * L, L)]
            for jj in range(L):
                j = i * L + jj
                qrow = qrow_v[j, pl.ds(0, L)]
                krow = krow_v[j, pl.ds(0, L)]
                kshift = krow.at[perm_k].get(mode='promise_in_bounds')
                eaj = ea_ch.at[jnp.full((L,), jj, jnp.int32)].get(
                    mode='promise_in_bounds')
                a = qrow + kshift + eaj * ccv
                a = jnp.where(a >= 0.0, a, a * 0.2)
                g_v[j] = jnp.exp(a)
            return 0

        lax.fori_loop(0, EBLK // L, chunk_body, 0)
        pltpu.async_copy(g_v, gE_hbm.at[pl.ds(eoff, EBLK), :], sem3)
        return 0

    lax.fori_loop(0, nblk, blk, 0)
    pltpu.make_async_copy(g_v, gE_hbm.at[pl.ds(0, EBLK), :], sem3).wait()


_HPC = HEADS // NC
_ROWS = NPAD // NS
_NROWS = 632
_NROWS_LAST = N - (NS - 1) * _NROWS


_BMAX = NBLK_TOTAL // NS + 2
_UNR = 6


@functools.partial(
    pl.kernel, mesh=_mesh,
    out_type=(jax.ShapeDtypeStruct((HEADS, NPAD, OUT), jnp.float32),
              jax.ShapeDtypeStruct((HEADS * NPAD,), jnp.float32)),
    scratch_types=[
        pltpu.VMEM((EBLK,), jnp.int32),
        pltpu.VMEM((EBLK,), jnp.int32),
        pltpu.VMEM((EBLK,), jnp.int32),
        pltpu.VMEM((EBLK,), jnp.int32),
        pltpu.VMEM((EBLK,), jnp.int32),
        pltpu.VMEM((EBLK,), jnp.int32),
        pltpu.VMEM((EBLK,), jnp.int32),
        pltpu.VMEM((EBLK,), jnp.int32),
        pltpu.VMEM((EBLK, L), jnp.float32),
        pltpu.VMEM((EBLK,), jnp.float32),
        pltpu.VMEM((EBLK,), jnp.float32),
        pltpu.VMEM((EBLK, OUT), jnp.float32),
        pltpu.VMEM((EBLK, OUT), jnp.float32),
        pltpu.VMEM_SHARED((N, OUT), jnp.float32),
        pltpu.VMEM_SHARED((NPAD,), jnp.float32),
        pltpu.SemaphoreType.DMA,
        pltpu.SemaphoreType.DMA,
        pltpu.SemaphoreType.DMA,
        pltpu.SemaphoreType.DMA,
        pltpu.SemaphoreType.DMA,
        pltpu.SemaphoreType.DMA,
        pltpu.SemaphoreType.DMA,
        pltpu.SemaphoreType.DMA,
    ],
)
def _pass_b(dst2_hbm, sidx2_hbm, gE_hbm, xrel8_hbm,
            num8_hbm, den8_hbm,
            dm0, dm1, dm2, sm0, sm1, sm2, ix0, ix1,
            g_v, d0_v, d1_v, x0_v, x1_v,
            accn, accd,
            smm0, smm1, smm2, smg0, smg1, sml, smsc0, smsc1):
    c = lax.axis_index("c")
    s = lax.axis_index("s")
    nblk = jnp.where(s < NBLK_TOTAL - (NBLK_TOTAL // NS) * NS,
                     NBLK_TOTAL // NS + 1, NBLK_TOTAL // NS)
    rbase = s * _ROWS
    lane = lax.iota(jnp.int32, L)

    DM = (dm0, dm1, dm2)
    SM = (sm0, sm1, sm2)
    IX = (ix0, ix1)
    D_ = (d0_v, d1_v)
    X_ = (x0_v, x1_v)
    SMM = (smm0, smm1, smm2)
    SMG = (smg0, smg1)
    SSC = (smsc0, smsc1)

    def issue_meta(b, m):
        row = b * NS + s
        pltpu.async_copy(dst2_hbm.at[row], DM[m], SMM[m])
        pltpu.async_copy(sidx2_hbm.at[row], SM[m], SMM[m])

    def wait_meta(m):
        pltpu.make_async_copy(dst2_hbm.at[0], DM[m], SMM[m]).wait()
        pltpu.make_async_copy(sidx2_hbm.at[0], SM[m], SMM[m]).wait()

    def issue_gather(xs, i):
        pltpu.async_copy(xrel8_hbm.at[IX[i]], X_[xs], SMG[xs])

    def wait_gather(xs, i):
        pltpu.make_async_copy(xrel8_hbm.at[IX[i]], X_[xs], SMG[xs]).wait()

    def issue_g(b):
        eoff = (b * NS + s) * EBLK
        pltpu.async_copy(gE_hbm.at[pl.ds(eoff, EBLK), :], g_v, sml)

    def wait_g():
        pltpu.make_async_copy(gE_hbm.at[pl.ds(0, EBLK), :], g_v, sml).wait()

    def issue_sc(xs, m):
        pltpu.async_copy(X_[xs], accn.at[DM[m]], SSC[xs], add=True)
        pltpu.async_copy(D_[xs], accd.at[DM[m]], SSC[xs], add=True)

    def wait_sc(xs):
        pltpu.make_async_copy(X_[xs], accn.at[DM[0]], SSC[xs]).wait()
        pltpu.make_async_copy(D_[xs], accd.at[DM[0]], SSC[xs]).wait()

    def comp_idx(i, m, h):
        for k in range(EBLK // L):
            sl = pl.ds(k * L, L)
            IX[i][sl] = SM[m][sl] + h

    def zero_x0():
        def zr(i, _):
            for k in range(OUT // L):
                x0_v[i, pl.ds(k * L, L)] = jnp.zeros((L,), jnp.float32)
            return 0
        lax.fori_loop(0, EBLK, zr, 0)

    def phase(p, _):
        h = c * _HPC + p
        hidx = jnp.full((L,), h, jnp.int32)

        zero_x0()
        for k in range(EBLK // L):
            d0_v[pl.ds(k * L, L)] = jnp.zeros((L,), jnp.float32)

        @pl.when(s < NS - 1)
        def _():
            nb = s * _NROWS
            for k in range(_NROWS // EBLK):
                pltpu.sync_copy(x0_v, accn.at[pl.ds(nb + k * EBLK, EBLK), :])
            pltpu.sync_copy(x0_v.at[pl.ds(0, _NROWS % EBLK), :],
                            accn.at[pl.ds(nb + (_NROWS // EBLK) * EBLK,
                                          _NROWS % EBLK), :])

        @pl.when(s == NS - 1)
        def _():
            nb = (NS - 1) * _NROWS
            for k in range(_NROWS_LAST // EBLK):
                pltpu.sync_copy(x0_v, accn.at[pl.ds(nb + k * EBLK, EBLK), :])
            pltpu.sync_copy(x0_v.at[pl.ds(0, _NROWS_LAST % EBLK), :],
                            accn.at[pl.ds(nb + (_NROWS_LAST // EBLK) * EBLK,
                                          _NROWS_LAST % EBLK), :])

        for k in range(_ROWS // EBLK):
            pltpu.sync_copy(d0_v, accd.at[pl.ds(rbase + k * EBLK, EBLK)])
        plsc.subcore_barrier()

        issue_meta(0, 0)
        issue_meta(1, 1)
        issue_g(0)
        wait_meta(0)
        comp_idx(0, 0, h)
        issue_gather(0, 0)

        def six(b6, _):
            for u in range(_UNR):
                b = b6 * _UNR + u
                xs = u % 2
                m = u % 3

                @pl.when(b < nblk)
                def _():
                    @pl.when(b >= 1)
                    def _():
                        wait_sc(xs ^ 1)

                    @pl.when(b + 1 < nblk)
                    def _():
                        wait_meta((u + 1) % 3)
                        comp_idx((u + 1) % 2, (u + 1) % 3, h)
                        issue_gather(xs ^ 1, (u + 1) % 2)

                    @pl.when(b + 2 < nblk)
                    def _():
                        issue_meta(b + 2, (u + 2) % 3)

                    wait_gather(xs, u % 2)
                    wait_g()

                    def chunk(i, _):
                        dv = jnp.zeros((L,), jnp.float32)
                        for jj in range(L):
                            j = i * L + jj
                            gj = g_v[j].at[hidx].get(mode='promise_in_bounds')
                            dv = jnp.where(lane == jj, gj, dv)
                            for kk in range(OUT // L):
                                sl = pl.ds(kk * L, L)
                                X_[xs][j, sl] = X_[xs][j, sl] * gj
                        D_[xs][pl.ds(i * L, L)] = dv
                        return 0

                    lax.fori_loop(0, EBLK // L, chunk, 0)
                    issue_sc(xs, m)

                    @pl.when(b + 1 < nblk)
                    def _():
                        issue_g(b + 1)
            return 0

        lax.fori_loop(0, (_BMAX + _UNR - 1) // _UNR, six, 0)

        @pl.when(nblk % 2 == 1)
        def _():
            wait_sc(0)

        @pl.when(nblk % 2 == 0)
        def _():
            wait_sc(1)

        plsc.subcore_barrier()

        @pl.when(s < NS - 1)
        def _():
            nb = s * _NROWS
            pltpu.sync_copy(accn.at[pl.ds(nb, _NROWS), :],
                            num8_hbm.at[h, pl.ds(nb, _NROWS), :])

        @pl.when(s == NS - 1)
        def _():
            nb = (NS - 1) * _NROWS
            pltpu.sync_copy(accn.at[pl.ds(nb, _NROWS_LAST), :],
                            num8_hbm.at[h, pl.ds(nb, _NROWS_LAST), :])
            zero_x0()
            pltpu.sync_copy(x0_v, num8_hbm.at[h, pl.ds(N, EBLK), :])
            pltpu.sync_copy(x0_v.at[pl.ds(0, NPAD - N - EBLK), :],
                            num8_hbm.at[h, pl.ds(N + EBLK, NPAD - N - EBLK), :])

        pltpu.sync_copy(accd.at[pl.ds(rbase, _ROWS)],
                        den8_hbm.at[pl.ds(h * NPAD + rbase, _ROWS)])
        plsc.subcore_barrier()
        return 0

    lax.fori_loop(0, _HPC, phase, 0)


def _layer(x, src, dst, et, ea, W, q, k, We, e, b):
    xrel = _mm_xrel(x, W)
    qk_cat = jnp.concatenate([q, k], axis=1)
    Wqk = _mm_wqk(W, qk_cat)
    qktab = _mm_qktab(x, Wqk).reshape(R * N, 128)
    cc = _cc_vec(We, e)
    gT, sidx8 = _pass_a(src, dst, et, ea, qktab, cc)
    xrel8 = xrel.reshape(R * N * HEADS, OUT)
    num8, den8p = _pass_b(dst.reshape(NBLK_TOTAL, EBLK),
                          sidx8.reshape(NBLK_TOTAL, EBLK), gT, xrel8)
    return num8, den8p.reshape(HEADS, NPAD)


def kernel(x, edge_index, edge_type, edge_attr, batch,
           W1, q1, k1, We1, e1, b1,
           W2, q2, k2, We2, e2, b2,
           W3, q3, k3, We3, e3, b3,
           lin_w, lin_b):
    src = edge_index[0]
    dst = edge_index[1]
    ea = edge_attr[:, 0]
    batch3 = jnp.pad(batch, (0, NPAD - N),
                     constant_values=G).reshape(NPAD // 1024, 1, 1024)

    num8, den8 = _layer(x, src, dst, edge_type, ea, W1, q1, k1, We1, e1, b1)
    h = _normalize(num8, den8, b1.reshape(HEADS, OUT))[:N]
    num8, den8 = _layer(h, src, dst, edge_type, ea, W2, q2, k2, We2, e2, b2)
    h = _normalize(num8, den8, b2.reshape(HEADS, OUT))[:N]
    num8, den8 = _layer(h, src, dst, edge_type, ea, W3, q3, k3, We3, e3, b3)
    return _final(num8, den8, b3.reshape(HEADS, OUT), lin_w,
                  lin_b.reshape(1, ACT), batch3)
